# baseline (speedup 1.0000x reference)
import jax
import jax.numpy as jnp
from jax.experimental import pallas as pl
from jax.experimental.pallas import tpu as pltpu

B = 32
TV = TT = TA = 50
PER = TV + TT + TA
DV, DT, DA = 512, 768, 128
D = 64
H, C = 4, 16
E = B * PER * PER
K = int(E * 0.5)
G = 8
NSTEP = B // G


def _fused(vis_ref, txt_ref, aud_ref, xm_ref,
           w1_ref, w2_ref, aa_ref, bc_ref, eb_ref,
           out_ref,
           biasD_s, score_s, h_s, ex_s):
    f32 = jnp.float32
    pid = pl.program_id(0)

    @pl.when(pid == 0)
    def _build_bias():
        ii = jax.lax.broadcasted_iota(jnp.int32, (PER, PER), 0)
        jj = jax.lax.broadcasted_iota(jnp.int32, (PER, PER), 1)
        ti = jax.lax.rem(ii, TV)
        tj = jax.lax.rem(jj, TV)
        trel = jnp.where(tj > ti, 2, jnp.where(tj < ti, 0, 1))
        etype = trel * 9 + (ii // TV) * 3 + (jj // TV)
        for hh in range(H):
            acc = jnp.zeros((PER, PER), f32)
            for n in range(27):
                acc = acc + jnp.where(etype == n, eb_ref[n, hh], 0.0)
            biasD_s[hh] = acc

    @pl.when(pid < NSTEP)
    def _stage1():
        Wv1 = w1_ref[0:DV]
        Wt1 = w1_ref[DV:DV + DT]
        Wa1 = w1_ref[DV + DT:DV + DT + DA]
        Wv2 = w2_ref[0:D]
        Wt2 = w2_ref[D:2 * D]
        Wa2 = w2_ref[2 * D:3 * D]
        Wg = w2_ref[3 * D:4 * D]
        As = aa_ref[0:D]
        Ad = aa_ref[D:2 * D]

        def mlp(x, W1, brow1, W2, brow2):
            y = jnp.maximum(jnp.dot(x, W1, preferred_element_type=f32) + brow1, 0.0)
            return jnp.maximum(jnp.dot(y, W2, preferred_element_type=f32) + brow2, 0.0)

        for g in range(G):
            b = pid * G + g
            v = mlp(vis_ref[g], Wv1, bc_ref[0:1], Wv2, bc_ref[1:2])
            t = mlp(txt_ref[g], Wt1, bc_ref[2:3], Wt2, bc_ref[3:4])
            a = mlp(aud_ref[g], Wa1, bc_ref[4:5], Wa2, bc_ref[5:6])
            x = jnp.concatenate([v, t, a], axis=0) * xm_ref[g]
            h = jnp.dot(x, Wg, preferred_element_type=f32)
            h_s[b] = h.astype(jnp.bfloat16)
            asrc = jnp.dot(h, As, preferred_element_type=f32)
            adstT = jnp.transpose(jnp.dot(h, Ad, preferred_element_type=f32))

            score = jnp.zeros((PER, PER), f32)
            for hh in range(H):
                eh = asrc[:, hh:hh + 1] + adstT[hh:hh + 1, :] + biasD_s[hh]
                eh = jnp.where(eh >= 0, eh, 0.2 * eh)
                m = jnp.max(eh, axis=0, keepdims=True)
                ex = jnp.exp(eh - m)
                ex_s[hh, b] = ex.astype(jnp.bfloat16)
                den = jnp.sum(ex, axis=0, keepdims=True)
                score = score + ex / (den + 1e-16)
            score_s[b] = score * (1.0 / H)

    @pl.when(pid == NSTEP)
    def _prune_and_aggregate():
        bits = jax.lax.bitcast_convert_type(score_s[...], jnp.int32)

        def bit_step(i, prefix):
            cand = prefix | (jnp.int32(1) << (jnp.int32(30) - i))
            cnt = jnp.sum((bits >= cand).astype(jnp.int32))
            return jnp.where(cnt >= K, cand, prefix)

        prefix = jax.lax.fori_loop(0, 31, bit_step, jnp.int32(0))
        thr = jax.lax.bitcast_convert_type(prefix, f32)

        bf = jnp.bfloat16
        keep = (score_s[...] >= thr).astype(f32).astype(bf)
        h_aug = jnp.concatenate(
            [h_s[...], jnp.ones((B, PER, 1), bf)], axis=2)
        outs = []
        for hh in range(H):
            anum = ex_s[hh] * keep
            res = jax.lax.dot_general(
                anum, h_aug, (((1,), (1,)), ((0,), (0,))),
                preferred_element_type=f32)
            r = 1.0 / (res[:, :, D:D + 1] + 1e-16)
            outs.append(res[:, :, hh * C:(hh + 1) * C] * r)
        o = jnp.concatenate(outs, axis=2)
        out_ref[...] = jnp.where(o >= 0, o, 0.1 * o)


def kernel(vision, text, audio, v_mask, t_mask, a_mask,
           Wv1, bv1, Wv2, bv2, Wt1, bt1, Wt2, bt2, Wa1, ba1, Wa2, ba2,
           Wg, att_src, att_dst, edge_bias):
    f32 = jnp.float32
    xmask = jnp.concatenate([v_mask, t_mask, a_mask], axis=1).astype(f32)[:, :, None]
    w1 = jnp.concatenate([Wv1, Wt1, Wa1], axis=0)
    w2 = jnp.concatenate([Wv2, Wt2, Wa2, Wg], axis=0)
    eyeH = jnp.eye(H, dtype=f32)
    As = (att_src[:, :, None] * eyeH[:, None, :]).reshape(H * C, H)
    Ad = (att_dst[:, :, None] * eyeH[:, None, :]).reshape(H * C, H)
    aa = jnp.concatenate([As, Ad], axis=0)
    bc = jnp.stack([bv1, bv2, bt1, bt2, ba1, ba2], axis=0)

    full = lambda shape: pl.BlockSpec(shape, lambda i: (0,) * len(shape))
    stepb = lambda shape: pl.BlockSpec(
        shape, lambda i: (jnp.minimum(i, NSTEP - 1),) + (0,) * (len(shape) - 1))

    out = pl.pallas_call(
        _fused,
        grid=(NSTEP + 1,),
        in_specs=[
            stepb((G, TV, DV)), stepb((G, TT, DT)), stepb((G, TA, DA)),
            stepb((G, PER, 1)),
            full((DV + DT + DA, D)), full((4 * D, D)), full((2 * D, H)),
            full((6, D)),
            pl.BlockSpec(memory_space=pltpu.SMEM),
        ],
        out_specs=pl.BlockSpec((B, PER, D), lambda i: (0, 0, 0)),
        out_shape=jax.ShapeDtypeStruct((B, PER, D), f32),
        scratch_shapes=[
            pltpu.VMEM((H, PER, PER), f32),
            pltpu.VMEM((B, PER, PER), f32),
            pltpu.VMEM((B, PER, D), jnp.bfloat16),
            pltpu.VMEM((H, B, PER, PER), jnp.bfloat16),
        ],
    )(vision, text, audio, xmask, w1, w2, aa, bc, edge_bias)

    return out.reshape(B * PER, H * C)


# speedup vs baseline: 1.2059x; 1.2059x over previous
import jax
import jax.numpy as jnp
from jax.experimental import pallas as pl
from jax.experimental.pallas import tpu as pltpu

B = 32
TV = TT = TA = 50
PER = TV + TT + TA
D = 64
H, C = 4, 16
E = B * PER * PER
K = int(E * 0.5)
G = 8
NSTEP = B // G


def _fused(vis_ref, txt_ref, aud_ref, xm_ref,
           Wv1_ref, bv1_ref, Wv2_ref, bv2_ref,
           Wt1_ref, bt1_ref, Wt2_ref, bt2_ref,
           Wa1_ref, ba1_ref, Wa2_ref, ba2_ref,
           Wg_ref, As_ref, Ad_ref, eb_ref,
           out_ref,
           biasD_s, score_s, cnt_s, h_s, ex_s):
    f32 = jnp.float32
    pid = pl.program_id(0)

    @pl.when(pid == 0)
    def _build_bias():
        cnt_s[...] = jnp.zeros((B // 4, 152, 4 * 160), f32)
        ii = jax.lax.broadcasted_iota(jnp.int32, (PER, PER), 0)
        jj = jax.lax.broadcasted_iota(jnp.int32, (PER, PER), 1)
        ti = jax.lax.rem(ii, TV)
        tj = jax.lax.rem(jj, TV)
        trel = jnp.where(tj > ti, 2, jnp.where(tj < ti, 0, 1))
        etype = trel * 9 + (ii // TV) * 3 + (jj // TV)
        for hh in range(H):
            acc = jnp.zeros((PER, PER), f32)
            for n in range(27):
                acc = acc + jnp.where(etype == n, eb_ref[n, hh], 0.0)
            biasD_s[hh] = acc

    @pl.when(pid < NSTEP)
    def _stage1():
        def mlp(x, W1, b1, W2, b2):
            y = jnp.maximum(jnp.dot(x, W1, preferred_element_type=f32) + b1, 0.0)
            return jnp.maximum(jnp.dot(y, W2, preferred_element_type=f32) + b2, 0.0)

        for g in range(G):
            b = pid * G + g
            v = mlp(vis_ref[g], Wv1_ref[...], bv1_ref[...], Wv2_ref[...], bv2_ref[...])
            t = mlp(txt_ref[g], Wt1_ref[...], bt1_ref[...], Wt2_ref[...], bt2_ref[...])
            a = mlp(aud_ref[g], Wa1_ref[...], ba1_ref[...], Wa2_ref[...], ba2_ref[...])
            x = jnp.concatenate([v, t, a], axis=0) * xm_ref[g]
            h = jnp.dot(x, Wg_ref[...], preferred_element_type=f32)
            h_s[b] = h.astype(jnp.bfloat16)
            asrc = jnp.dot(h, As_ref[...], preferred_element_type=f32)
            adstT = jnp.transpose(jnp.dot(h, Ad_ref[...], preferred_element_type=f32))

            score = jnp.zeros((PER, PER), f32)
            for hh in range(H):
                eh = asrc[:, hh:hh + 1] + adstT[hh:hh + 1, :] + biasD_s[hh]
                eh = jnp.where(eh >= 0, eh, 0.2 * eh)
                m = jnp.max(eh, axis=0, keepdims=True)
                ex = jnp.exp(eh - m)
                ex_s[hh, b] = ex.astype(jnp.bfloat16)
                den = jnp.sum(ex, axis=0, keepdims=True)
                score = score + ex / (den + 1e-16)
            score = score * (1.0 / H)
            score_s[b] = score
            cnt_s[pid * 2 + g // 4, 0:PER,
                  (g % 4) * PER:(g % 4) * PER + PER] = score

    @pl.when(pid == NSTEP)
    def _prune_and_aggregate():
        bits = jax.lax.bitcast_convert_type(
            cnt_s[...], jnp.int32).reshape(8 * 152, 4 * 160)
        ones_row = jnp.ones((1, 8 * 152), f32)

        def bit_step(i, prefix):
            cand = prefix | (jnp.int32(1) << (jnp.int32(30) - i))
            mask = (bits >= cand).astype(f32)
            psum = jax.lax.dot_general(
                ones_row, mask, (((1,), (0,)), ((), ())),
                preferred_element_type=f32)
            cnt = jnp.sum(psum)
            return jnp.where(cnt >= f32(K), cand, prefix)

        prefix = jax.lax.fori_loop(0, 31, bit_step, jnp.int32(0))
        thr = jax.lax.bitcast_convert_type(prefix, f32)

        bf = jnp.bfloat16
        keep = (score_s[...] >= thr).astype(f32).astype(bf)
        h_aug = jnp.concatenate(
            [h_s[...], jnp.ones((B, PER, 1), bf)], axis=2)
        outs = []
        for hh in range(H):
            anum = ex_s[hh] * keep
            res = jax.lax.dot_general(
                anum, h_aug, (((1,), (1,)), ((0,), (0,))),
                preferred_element_type=f32)
            r = 1.0 / (res[:, :, D:D + 1] + 1e-16)
            outs.append(res[:, :, hh * C:(hh + 1) * C] * r)
        o = jnp.concatenate(outs, axis=2)
        out_ref[...] = jnp.where(o >= 0, o, 0.1 * o)


def kernel(vision, text, audio, v_mask, t_mask, a_mask,
           Wv1, bv1, Wv2, bv2, Wt1, bt1, Wt2, bt2, Wa1, ba1, Wa2, ba2,
           Wg, att_src, att_dst, edge_bias):
    f32 = jnp.float32
    xmask = jnp.concatenate([v_mask, t_mask, a_mask], axis=1).astype(f32)[:, :, None]
    eyeH = jnp.eye(H, dtype=f32)
    As = (att_src[:, :, None] * eyeH[:, None, :]).reshape(H * C, H)
    Ad = (att_dst[:, :, None] * eyeH[:, None, :]).reshape(H * C, H)

    full = lambda shape: pl.BlockSpec(shape, lambda i: (0,) * len(shape))
    stepb = lambda shape: pl.BlockSpec(
        shape, lambda i: (jnp.minimum(i, NSTEP - 1),) + (0,) * (len(shape) - 1))

    out = pl.pallas_call(
        _fused,
        grid=(NSTEP + 1,),
        in_specs=[
            stepb((G, TV, 512)), stepb((G, TT, 768)), stepb((G, TA, 128)),
            stepb((G, PER, 1)),
            full((512, D)), full((D,)), full((D, D)), full((D,)),
            full((768, D)), full((D,)), full((D, D)), full((D,)),
            full((128, D)), full((D,)), full((D, D)), full((D,)),
            full((D, D)), full((D, H)), full((D, H)),
            pl.BlockSpec(memory_space=pltpu.SMEM),
        ],
        out_specs=pl.BlockSpec((B, PER, D), lambda i: (0, 0, 0)),
        out_shape=jax.ShapeDtypeStruct((B, PER, D), f32),
        scratch_shapes=[
            pltpu.VMEM((H, PER, PER), f32),
            pltpu.VMEM((B, PER, PER), f32),
            pltpu.VMEM((B // 4, 152, 4 * 160), f32),
            pltpu.VMEM((B, PER, D), jnp.bfloat16),
            pltpu.VMEM((H, B, PER, PER), jnp.bfloat16),
        ],
    )(vision, text, audio, xmask,
      Wv1, bv1, Wv2, bv2, Wt1, bt1, Wt2, bt2, Wa1, ba1, Wa2, ba2,
      Wg, As, Ad, edge_bias)

    return out.reshape(B * PER, H * C)


# speedup vs baseline: 1.2970x; 1.0755x over previous
import jax
import jax.numpy as jnp
from jax.experimental import pallas as pl
from jax.experimental.pallas import tpu as pltpu

B = 32
TV = TT = TA = 50
PER = TV + TT + TA
D = 64
H, C = 4, 16
E = B * PER * PER
K = int(E * 0.5)
G = 8
NSTEP = B // G


def _fused(vis_ref, txt_ref, aud_ref, xm_ref,
           Wv1_ref, bv1_ref, Wv2_ref, bv2_ref,
           Wt1_ref, bt1_ref, Wt2_ref, bt2_ref,
           Wa1_ref, ba1_ref, Wa2_ref, ba2_ref,
           Wg_ref, Asd_ref, eb_ref,
           out_ref,
           biasD_s, score_s, cnt_s, h_s, ex_s):
    f32 = jnp.float32
    pid = pl.program_id(0)

    @pl.when(pid == 0)
    def _build_bias():
        cnt_s[...] = jnp.zeros((B // 4, 152, 4 * 160), f32)
        ii = jax.lax.broadcasted_iota(jnp.int32, (PER, PER), 0)
        jj = jax.lax.broadcasted_iota(jnp.int32, (PER, PER), 1)
        ti = jax.lax.rem(ii, TV)
        tj = jax.lax.rem(jj, TV)
        trel = jnp.where(tj > ti, 2, jnp.where(tj < ti, 0, 1))
        etype = trel * 9 + (ii // TV) * 3 + (jj // TV)
        for hh in range(H):
            acc = jnp.zeros((PER, PER), f32)
            for n in range(27):
                acc = acc + jnp.where(etype == n, eb_ref[n, hh], 0.0)
            biasD_s[hh] = acc

    @pl.when(pid < NSTEP)
    def _stage1():
        def mlp(x, W1, b1, W2, b2):
            y = jnp.maximum(jnp.dot(x, W1, preferred_element_type=f32) + b1, 0.0)
            return jnp.maximum(jnp.dot(y, W2, preferred_element_type=f32) + b2, 0.0)

        mats = []
        for (ref, W1, b1, W2, b2, col) in (
                (vis_ref, Wv1_ref, bv1_ref, Wv2_ref, bv2_ref, 0),
                (txt_ref, Wt1_ref, bt1_ref, Wt2_ref, bt2_ref, 1),
                (aud_ref, Wa1_ref, ba1_ref, Wa2_ref, ba2_ref, 2)):
            xm = mlp(ref[...], W1[...], b1[...], W2[...], b2[...])
            xm = xm * xm_ref[:, col:col + 1]
            hm = jnp.dot(xm, Wg_ref[...], preferred_element_type=f32)
            am = jnp.dot(hm, Asd_ref[...], preferred_element_type=f32)
            mats.append((hm, am))

        for g in range(G):
            b = pid * G + g
            sl = slice(g * TV, (g + 1) * TV)
            h = jnp.concatenate([mats[0][0][sl], mats[1][0][sl],
                                 mats[2][0][sl]], axis=0)
            h_s[b] = h.astype(jnp.bfloat16)
            ad = jnp.concatenate([mats[0][1][sl], mats[1][1][sl],
                                  mats[2][1][sl]], axis=0)
            asrc = ad[:, 0:H]
            adstT = jnp.transpose(ad[:, H:2 * H])

            score = jnp.zeros((PER, PER), f32)
            for hh in range(H):
                eh = asrc[:, hh:hh + 1] + adstT[hh:hh + 1, :] + biasD_s[hh]
                eh = jnp.where(eh >= 0, eh, 0.2 * eh)
                m = jnp.max(eh, axis=0, keepdims=True)
                ex = jnp.exp(eh - m)
                ex_s[hh, b] = ex.astype(jnp.bfloat16)
                den = jnp.sum(ex, axis=0, keepdims=True)
                score = score + ex / (den + 1e-16)
            score = score * (1.0 / H)
            score_s[b] = score
            cnt_s[pid * 2 + g // 4, 0:PER,
                  (g % 4) * PER:(g % 4) * PER + PER] = score

    @pl.when(pid == NSTEP)
    def _prune_and_aggregate():
        bits = jax.lax.bitcast_convert_type(
            cnt_s[...], jnp.int32).reshape(8 * 152, 4 * 160)
        ones_row = jnp.ones((1, 8 * 152), f32)

        def bit_step(i, prefix):
            cand = prefix | (jnp.int32(1) << (jnp.int32(30) - i))
            mask = (bits >= cand).astype(f32)
            psum = jax.lax.dot_general(
                ones_row, mask, (((1,), (0,)), ((), ())),
                preferred_element_type=f32)
            cnt = jnp.sum(psum)
            return jnp.where(cnt >= f32(K), cand, prefix)

        prefix = jax.lax.fori_loop(0, 31, bit_step, jnp.int32(0))
        thr = jax.lax.bitcast_convert_type(prefix, f32)

        bf = jnp.bfloat16
        keep = (score_s[...] >= thr).astype(f32).astype(bf)
        h_aug = jnp.concatenate(
            [h_s[...], jnp.ones((B, PER, 1), bf)], axis=2)
        outs = []
        for hh in range(H):
            anum = ex_s[hh] * keep
            res = jax.lax.dot_general(
                anum, h_aug, (((1,), (1,)), ((0,), (0,))),
                preferred_element_type=f32)
            r = 1.0 / (res[:, :, D:D + 1] + 1e-16)
            outs.append(res[:, :, hh * C:(hh + 1) * C] * r)
        o = jnp.concatenate(outs, axis=2)
        out_ref[...] = jnp.where(o >= 0, o, 0.1 * o)


def kernel(vision, text, audio, v_mask, t_mask, a_mask,
           Wv1, bv1, Wv2, bv2, Wt1, bt1, Wt2, bt2, Wa1, ba1, Wa2, ba2,
           Wg, att_src, att_dst, edge_bias):
    f32 = jnp.float32
    xmask = jnp.stack([v_mask.reshape(-1), t_mask.reshape(-1),
                       a_mask.reshape(-1)], axis=1).astype(f32)
    eyeH = jnp.eye(H, dtype=f32)
    As = (att_src[:, :, None] * eyeH[:, None, :]).reshape(H * C, H)
    Ad = (att_dst[:, :, None] * eyeH[:, None, :]).reshape(H * C, H)
    Asd = jnp.concatenate([As, Ad], axis=1)

    full = lambda shape: pl.BlockSpec(shape, lambda i: (0,) * len(shape))
    stepb = lambda shape: pl.BlockSpec(
        shape, lambda i: (jnp.minimum(i, NSTEP - 1),) + (0,) * (len(shape) - 1))

    out = pl.pallas_call(
        _fused,
        grid=(NSTEP + 1,),
        in_specs=[
            stepb((G * TV, 512)), stepb((G * TT, 768)), stepb((G * TA, 128)),
            stepb((G * TV, 3)),
            full((512, D)), full((D,)), full((D, D)), full((D,)),
            full((768, D)), full((D,)), full((D, D)), full((D,)),
            full((128, D)), full((D,)), full((D, D)), full((D,)),
            full((D, D)), full((D, 2 * H)),
            pl.BlockSpec(memory_space=pltpu.SMEM),
        ],
        out_specs=pl.BlockSpec((B, PER, D), lambda i: (0, 0, 0)),
        out_shape=jax.ShapeDtypeStruct((B, PER, D), f32),
        scratch_shapes=[
            pltpu.VMEM((H, PER, PER), f32),
            pltpu.VMEM((B, PER, PER), f32),
            pltpu.VMEM((B // 4, 152, 4 * 160), f32),
            pltpu.VMEM((B, PER, D), jnp.bfloat16),
            pltpu.VMEM((H, B, PER, PER), jnp.bfloat16),
        ],
    )(vision.reshape(B * TV, 512), text.reshape(B * TT, 768),
      audio.reshape(B * TA, 128), xmask,
      Wv1, bv1, Wv2, bv2, Wt1, bt1, Wt2, bt2, Wa1, ba1, Wa2, ba2,
      Wg, Asd, edge_bias)

    return out.reshape(B * PER, H * C)
